# baseline (speedup 1.0000x reference)
import functools
import math
import jax
import jax.numpy as jnp
from jax import lax
from jax.experimental import pallas as pl
from jax.experimental.pallas import tpu as pltpu
from jax.experimental.pallas import tpu_sc as plsc

_M = 4
_G = 512
_HW = 1024
_NB = 32
_BLK = _G * _HW
_CH = 32
_NCHUNK = _G // _CH
_CHW = _CH * _HW
_STRIPS = _HW // 16
_INV_SCALE = 1.0 / math.sqrt(_G)
_NEG = -3.0e38

_mesh = plsc.VectorSubcoreMesh(
    core_axis_name="c", subcore_axis_name="s", num_cores=2, num_subcores=16
)


def _sc_body(x_hbm, q_hbm, buf_a, buf_b, mxv, ixv, si0, si1, so0, so1):
    w = lax.axis_index("s") * 2 + lax.axis_index("c")
    bufs = (buf_a, buf_b)
    sin = (si0, si1)
    sout = (so0, so1)
    lane = lax.iota(jnp.int32, 16)
    negv = jnp.full((16,), _NEG, jnp.float32)
    zi = jnp.zeros((16,), jnp.int32)
    onesv = jnp.full((16,), 1.0, jnp.float32)
    zerov = jnp.zeros((16,), jnp.float32)

    def init_body(j, c):
        off = pl.multiple_of(j * 16, 16)
        mxv[pl.ds(off, 16)] = negv
        ixv[pl.ds(off, 16)] = zi
        return c

    lax.fori_loop(0, _STRIPS, init_body, 0)

    handles = [pltpu.async_copy(x_hbm.at[w, pl.ds(0, _CHW)], buf_a, si0), None]
    for k in range(_NCHUNK):
        b = k & 1
        handles[b].wait()
        if k + 1 < _NCHUNK:
            nxt = (k + 1) & 1
            handles[nxt] = pltpu.async_copy(
                x_hbm.at[w, pl.ds((k + 1) * _CHW, _CHW)], bufs[nxt], sin[nxt]
            )
        buf = bufs[b]

        def strip_body(j, c, buf=buf, k=k):
            off = pl.multiple_of(j * 16, 16)
            m = mxv[pl.ds(off, 16)]
            i = ixv[pl.ds(off, 16)]
            for r in range(_CH):
                v = buf[pl.ds(off + r * _HW, 16)]
                gt = v > m
                m = jnp.where(gt, v, m)
                i = jnp.where(gt, jnp.full((16,), k * _CH + r, jnp.int32), i)
            mxv[pl.ds(off, 16)] = m
            ixv[pl.ds(off, 16)] = i
            return c

        lax.fori_loop(0, _STRIPS, strip_body, 0)

    def zero_body(t, c, buf=None):
        off = pl.multiple_of(t * 64, 16)
        for u in range(4):
            buf[pl.ds(off + u * 16, 16)] = zerov
        return c

    lax.fori_loop(0, _CHW // 64, functools.partial(zero_body, buf=buf_a), 0)
    lax.fori_loop(0, _CHW // 64, functools.partial(zero_body, buf=buf_b), 0)

    def scat(buf, k, val):
        def body(j, c):
            off = pl.multiple_of(j * 16, 16)
            iv = ixv[pl.ds(off, 16)]
            local = iv - (k * _CH)
            msk = (local >= 0) & (local < _CH)
            addr = local * _HW + off + lane
            addr = jnp.where(msk, addr, zi)
            plsc.store_scatter(buf, [addr], val, mask=msk)
            return c

        lax.fori_loop(0, _STRIPS, body, 0)

    out_h = [None, None]
    for k in range(_NCHUNK):
        b = k & 1
        if out_h[b] is not None:
            out_h[b].wait()
            scat(bufs[b], k - 2, zerov)
        scat(bufs[b], k, onesv)
        out_h[b] = pltpu.async_copy(
            bufs[b], q_hbm.at[w, pl.ds(k * _CHW, _CHW)], sout[b]
        )
    out_h[0].wait()
    out_h[1].wait()


_sc_quantize = functools.partial(
    pl.kernel,
    out_type=jax.ShapeDtypeStruct((_NB, _BLK), jnp.float32),
    mesh=_mesh,
    compiler_params=pltpu.CompilerParams(needs_layout_passes=False),
    scratch_types=[
        pltpu.VMEM((_CHW,), jnp.float32),
        pltpu.VMEM((_CHW,), jnp.float32),
        pltpu.VMEM((_HW,), jnp.float32),
        pltpu.VMEM((_HW,), jnp.int32),
        pltpu.SemaphoreType.DMA,
        pltpu.SemaphoreType.DMA,
        pltpu.SemaphoreType.DMA,
        pltpu.SemaphoreType.DMA,
    ],
)(_sc_body)


def _tc_body(x_ref, l_ref):
    l_ref[...] = x_ref[...] * _INV_SCALE


def _tc_scale(x3):
    blk = (2, _G, _HW)
    return pl.pallas_call(
        _tc_body,
        grid=(_NB // 2,),
        in_specs=[pl.BlockSpec(blk, lambda i: (i, 0, 0))],
        out_specs=pl.BlockSpec(blk, lambda i: (i, 0, 0)),
        out_shape=jax.ShapeDtypeStruct(x3.shape, x3.dtype),
    )(x3)


def kernel(logit, temperature):
    n, c, h, w = logit.shape
    g = c // _M
    x2 = logit.reshape(_NB, _BLK)
    q2 = _sc_quantize(x2)
    l3 = _tc_scale(logit.reshape(_NB, _G, _HW))
    return q2.reshape(n, c, h, w), l3.reshape(n, _M, g, h, w)
